# Initial kernel scaffold; baseline (speedup 1.0000x reference)
import jax
import jax.numpy as jnp
from jax.experimental import pallas as pl


def kernel(c_it, medicine_it, c_embeddings, m_embeddings, W1, b1, W2, att2, b2, Wl, hyperedge_attr):
    raise NotImplementedError("write your pallas kernel here")


# speedup vs baseline: 1574.5960x; 1574.5960x over previous
import functools

import jax
import jax.numpy as jnp
from jax.experimental import pallas as pl
from jax.experimental.pallas import tpu as pltpu
from jax.experimental.pallas import tpu_sc as plsc


_W = 128


def _sc_gather(c_table, c_idx, m_table, m_idx):
    nc = c_idx.shape[1]
    nm = m_idx.shape[1]
    dim = c_table.shape[1]
    wc = wm = _W
    mesh = plsc.VectorSubcoreMesh(core_axis_name="c", subcore_axis_name="s")

    @pl.kernel(
        out_type=(
            jax.ShapeDtypeStruct((nc, dim), c_table.dtype),
            jax.ShapeDtypeStruct((nm, dim), m_table.dtype),
        ),
        mesh=mesh,
    )
    def gather_kernel(c_hbm, ci_hbm, m_hbm, mi_hbm, o_dia, o_med):
        def body_c(i_vmem, o_vmem):
            pltpu.sync_copy(c_hbm.at[i_vmem.at[0]], o_vmem)

        pltpu.emit_pipeline(
            body_c,
            grid=(nc // wc,),
            in_specs=[pl.BlockSpec((1, wc), lambda i: (0, i))],
            out_specs=[pl.BlockSpec((wc, dim), lambda i: (i, 0))],
            core_axis_name=("c", "s"),
            dimension_semantics=(pltpu.PARALLEL,),
        )(ci_hbm, o_dia)

        def body_m(i_vmem, o_vmem):
            pltpu.sync_copy(m_hbm.at[i_vmem.at[0]], o_vmem)

        pltpu.emit_pipeline(
            body_m,
            grid=(nm // wm,),
            in_specs=[pl.BlockSpec((1, wm), lambda i: (0, i))],
            out_specs=[pl.BlockSpec((wm, dim), lambda i: (i, 0))],
            core_axis_name=("c", "s"),
            dimension_semantics=(pltpu.PARALLEL,),
        )(mi_hbm, o_med)

    return gather_kernel(c_table, c_idx, m_table, m_idx)


def _tc_body(nc, nm, dia_ref, med_ref, hat, w1, b1, w2, att_n, att_e, b2,
             wl_t, wl_b, o1, o2):
    f32 = jnp.float32
    dia = dia_ref[...][:nc]
    med = med_ref[...][:nm]
    xd = jnp.dot(dia, w2[...], preferred_element_type=f32)
    xm = jnp.dot(med, w2[...], preferred_element_type=f32)
    he = jnp.dot(hat[...], w2[...], preferred_element_type=f32)

    an = att_n[...]
    v = jnp.sum(he * att_e[...], axis=1, keepdims=True)
    ud = jnp.sum(xd * an, axis=1, keepdims=True)
    um = jnp.sum(xm * an, axis=1)

    lrelu = lambda x: jnp.where(x >= 0, x, 0.2 * x)
    a_dis = lrelu(ud + v)
    amat = lrelu(v + um[None, :])
    a_max = jnp.maximum(jnp.max(amat, axis=1, keepdims=True), a_dis)
    emat = jnp.exp(amat - a_max)
    p = jnp.exp(a_dis - a_max)
    ssum = jnp.sum(emat, axis=1, keepdims=True)
    denom = p + ssum + 1e-16
    g = jnp.dot(emat, xm, preferred_element_type=f32)
    ef = (p * xd + g) / denom * (1.0 / (nm + 1))
    sum1 = jnp.sum((p / denom) * ef, axis=0)[None, :]
    sum2 = jnp.sum((ssum / denom) * ef, axis=0)[None, :]

    sum_dia = jnp.sum(dia, axis=0)[None, :]
    sum_med = jnp.sum(med, axis=0)[None, :]
    t1 = jnp.dot(sum_dia, w1[...], preferred_element_type=f32) + nc * b1[...]
    t2 = jnp.dot(sum_med, w1[...], preferred_element_type=f32) + nm * b1[...]

    r1 = sum1 + nc * b2[...]
    r2 = sum2 * (1.0 / nc) + nm * b2[...]
    o1[...] = (jnp.dot(r1, wl_t[...], preferred_element_type=f32)
               + jnp.dot(t1, wl_b[...], preferred_element_type=f32))
    o2[...] = (jnp.dot(r2, wl_t[...], preferred_element_type=f32)
               + jnp.dot(t2, wl_b[...], preferred_element_type=f32))


def kernel(c_it, medicine_it, c_embeddings, m_embeddings, W1, b1, W2, att2,
           b2, Wl, hyperedge_attr):
    nc = c_it.shape[0]
    nm = medicine_it.shape[0]
    c = W2.shape[1]

    nc_pad = -(-nc // _W) * _W
    nm_pad = -(-nm // _W) * _W
    ci = jnp.zeros((1, nc_pad), jnp.int32).at[0, :nc].set(
        c_it.astype(jnp.int32))
    mi = jnp.zeros((1, nm_pad), jnp.int32).at[0, :nm].set(
        medicine_it.astype(jnp.int32))
    dia, med = _sc_gather(c_embeddings, ci, m_embeddings, mi)

    att_n = att2[:c].reshape(1, c)
    att_e = att2[c:].reshape(1, c)
    b1r = b1.reshape(1, c)
    b2r = b2.reshape(1, c)
    wl_t = Wl[:c]
    wl_b = Wl[c:]

    i1, i2 = pl.pallas_call(
        functools.partial(_tc_body, nc, nm),
        out_shape=(
            jax.ShapeDtypeStruct((1, c), jnp.float32),
            jax.ShapeDtypeStruct((1, c), jnp.float32),
        ),
    )(dia, med, hyperedge_attr, W1, b1r, W2, att_n, att_e, b2r, wl_t, wl_b)

    return i1.reshape(1, 1, c), i2.reshape(1, 1, c)


# speedup vs baseline: 1636.0590x; 1.0390x over previous
import functools

import jax
import jax.numpy as jnp
from jax.experimental import pallas as pl
from jax.experimental.pallas import tpu as pltpu
from jax.experimental.pallas import tpu_sc as plsc


_W = 128


def _sc_gather(c_table, c_idx, m_table, m_idx):
    nc = c_idx.shape[1]
    nm = m_idx.shape[1]
    dim = c_table.shape[1]
    wc = wm = _W
    mesh = plsc.VectorSubcoreMesh(core_axis_name="c", subcore_axis_name="s")

    @pl.kernel(
        out_type=(
            jax.ShapeDtypeStruct((nc, dim), c_table.dtype),
            jax.ShapeDtypeStruct((nm, dim), m_table.dtype),
        ),
        mesh=mesh,
    )
    def gather_kernel(c_hbm, ci_hbm, m_hbm, mi_hbm, o_dia, o_med):
        core = jax.lax.axis_index("c")

        @pl.when(core == 0)
        def _():
            def body_c(i_vmem, o_vmem):
                pltpu.sync_copy(c_hbm.at[i_vmem.at[0]], o_vmem)

            pltpu.emit_pipeline(
                body_c,
                grid=(nc // wc,),
                in_specs=[pl.BlockSpec((1, wc), lambda i: (0, i))],
                out_specs=[pl.BlockSpec((wc, dim), lambda i: (i, 0))],
                core_axis_name="s",
                dimension_semantics=(pltpu.PARALLEL,),
            )(ci_hbm, o_dia)

        @pl.when(core == 1)
        def _():
            def body_m(i_vmem, o_vmem):
                pltpu.sync_copy(m_hbm.at[i_vmem.at[0]], o_vmem)

            pltpu.emit_pipeline(
                body_m,
                grid=(nm // wm,),
                in_specs=[pl.BlockSpec((1, wm), lambda i: (0, i))],
                out_specs=[pl.BlockSpec((wm, dim), lambda i: (i, 0))],
                core_axis_name="s",
                dimension_semantics=(pltpu.PARALLEL,),
            )(mi_hbm, o_med)

    return gather_kernel(c_table, c_idx, m_table, m_idx)


def _tc_body(nc, nm, dia_ref, med_ref, hat, w1, b1, w2, att2, b2, wl,
             o1, o2):
    f32 = jnp.float32
    c = w2.shape[1]
    dia = dia_ref[...][:nc]
    med = med_ref[...][:nm]
    xd = jnp.dot(dia, w2[...], preferred_element_type=f32)
    xm = jnp.dot(med, w2[...], preferred_element_type=f32)
    he = jnp.dot(hat[...], w2[...], preferred_element_type=f32)

    att = att2[...]
    an = att[:c][None, :]
    ae = att[c:][None, :]
    b1v = b1[...][None, :]
    b2v = b2[...][None, :]
    wl_t = wl[...][:c]
    wl_b = wl[...][c:]
    v = jnp.sum(he * ae, axis=1, keepdims=True)
    ud = jnp.sum(xd * an, axis=1, keepdims=True)
    um = jnp.sum(xm * an, axis=1)

    lrelu = lambda x: jnp.where(x >= 0, x, 0.2 * x)
    a_dis = lrelu(ud + v)
    amat = lrelu(v + um[None, :])
    a_max = jnp.maximum(jnp.max(amat, axis=1, keepdims=True), a_dis)
    emat = jnp.exp(amat - a_max)
    p = jnp.exp(a_dis - a_max)
    ssum = jnp.sum(emat, axis=1, keepdims=True)
    denom = p + ssum + 1e-16
    g = jnp.dot(emat, xm, preferred_element_type=f32)
    ef = (p * xd + g) / denom * (1.0 / (nm + 1))
    sum1 = jnp.sum((p / denom) * ef, axis=0)[None, :]
    sum2 = jnp.sum((ssum / denom) * ef, axis=0)[None, :]

    sum_dia = jnp.sum(dia, axis=0)[None, :]
    sum_med = jnp.sum(med, axis=0)[None, :]
    t1 = jnp.dot(sum_dia, w1[...], preferred_element_type=f32) + nc * b1v
    t2 = jnp.dot(sum_med, w1[...], preferred_element_type=f32) + nm * b1v

    r1 = sum1 + nc * b2v
    r2 = sum2 * (1.0 / nc) + nm * b2v
    o1[...] = (jnp.dot(r1, wl_t, preferred_element_type=f32)
               + jnp.dot(t1, wl_b, preferred_element_type=f32))
    o2[...] = (jnp.dot(r2, wl_t, preferred_element_type=f32)
               + jnp.dot(t2, wl_b, preferred_element_type=f32))


def kernel(c_it, medicine_it, c_embeddings, m_embeddings, W1, b1, W2, att2,
           b2, Wl, hyperedge_attr):
    nc = c_it.shape[0]
    nm = medicine_it.shape[0]
    c = W2.shape[1]

    nc_pad = -(-nc // _W) * _W
    nm_pad = -(-nm // _W) * _W
    ci = jnp.zeros((1, nc_pad), jnp.int32).at[0, :nc].set(
        c_it.astype(jnp.int32))
    mi = jnp.zeros((1, nm_pad), jnp.int32).at[0, :nm].set(
        medicine_it.astype(jnp.int32))
    dia, med = _sc_gather(c_embeddings, ci, m_embeddings, mi)

    i1, i2 = pl.pallas_call(
        functools.partial(_tc_body, nc, nm),
        out_shape=(
            jax.ShapeDtypeStruct((1, c), jnp.float32),
            jax.ShapeDtypeStruct((1, c), jnp.float32),
        ),
    )(dia, med, hyperedge_attr, W1, b1, W2, att2, b2, Wl)

    return i1.reshape(1, 1, c), i2.reshape(1, 1, c)
